# Initial kernel scaffold; baseline (speedup 1.0000x reference)
import jax
import jax.numpy as jnp
from jax.experimental import pallas as pl


def kernel(atom_list, bond_list, atom_degree_list, bond_degree_list, atom_mask, params):
    raise NotImplementedError("write your pallas kernel here")


# speedup vs baseline: 20.5353x; 20.5353x over previous
import functools

import jax
import jax.numpy as jnp
from jax.experimental import pallas as pl
from jax.experimental.pallas import tpu as pltpu

MB = 8


def _leaky(x):
    return jnp.where(x >= 0, x, 0.01 * x)


def _elu(x):
    return jnp.where(x > 0, x, jnp.exp(x) - 1.0)


def _bmm(a, b):
    return jax.lax.dot_general(
        a, b, (((2,), (1,)), ((0,), (0,))),
        preferred_element_type=jnp.float32)


def _mm(a, b):
    return jnp.dot(a, b, preferred_element_type=jnp.float32)


def _gru(x, h, wxr, wxz, wxn, whr, whz, whn, bxr, bxz, bxn, bhr, bhz, bhn):
    r = jax.nn.sigmoid(_mm(x, wxr) + bxr + _mm(h, whr) + bhr)
    z = jax.nn.sigmoid(_mm(x, wxz) + bxz + _mm(h, whz) + bhz)
    n = jnp.tanh(_mm(x, wxn) + bxn + r * (_mm(h, whn) + bhn))
    return (1.0 - z) * n + z * h


def _fused_body(
    atom_ref, bond_ref, aidx_ref, bidx_ref, mask_ref,
    wf_ref, bf_ref, wna_ref, wnb_ref, bn_ref,
    a0w1_ref, a0w2_ref, a0b_ref, a1w1_ref, a1w2_ref, a1b_ref,
    t0w_ref, t0b_ref, t1w_ref, t1b_ref,
    g0xr_ref, g0xz_ref, g0xn_ref, g0hr_ref, g0hz_ref, g0hn_ref,
    g0bxr_ref, g0bxz_ref, g0bxn_ref, g0bhr_ref, g0bhz_ref, g0bhn_ref,
    g1xr_ref, g1xz_ref, g1xn_ref, g1hr_ref, g1hz_ref, g1hn_ref,
    g1bxr_ref, g1bxz_ref, g1bxn_ref, g1bhr_ref, g1bhz_ref, g1bhn_ref,
    mw1_ref, mw2_ref, mb_ref, mtw_ref, mtb_ref,
    mgxr_ref, mgxz_ref, mgxn_ref, mghr_ref, mghz_ref, mghn_ref,
    mgbxr_ref, mgbxz_ref, mgbxn_ref, mgbhr_ref, mgbhz_ref, mgbhn_ref,
    ow_ref, ob_ref,
    af_out_ref, pred_out_ref, molfeat_out_ref,
    *, mb, L, NBR, NB,
):
    R = mb * L
    atom = atom_ref[...].reshape(R, atom_ref.shape[-1])
    bond = bond_ref[...].reshape(mb * NB, bond_ref.shape[-1])
    aidx = aidx_ref[...]
    bidx = bidx_ref[...]
    mask = mask_ref[...]

    af = _leaky(_mm(atom, wf_ref[...]) + bf_ref[...])
    P = _mm(atom, wna_ref[...])
    Q = _mm(bond, wnb_ref[...])
    P3 = P.reshape(mb, L, 200)
    Q3 = Q.reshape(mb, NB, 200)

    iota_a = jax.lax.broadcasted_iota(jnp.int32, (1, L), 1)
    iota_b = jax.lax.broadcasted_iota(jnp.int32, (1, NB), 1)

    oha = []
    nf = []
    s2 = []
    amask = []
    smask = []
    for j in range(NBR):
        ij = aidx[:, j:j + 1]
        oha_j = (ij == iota_a).astype(jnp.float32)
        ohb_j = (bidx[:, j:j + 1] == iota_b).astype(jnp.float32)
        nfP = _bmm(oha_j.reshape(mb, L, L), P3).reshape(R, 200)
        nfQ = _bmm(ohb_j.reshape(mb, L, NB), Q3).reshape(R, 200)
        nf_j = _leaky(nfP + nfQ + bn_ref[...])
        pad = (ij == (L - 1))
        oha.append(oha_j)
        nf.append(nf_j)
        s2.append(_mm(nf_j, a0w2_ref[...]))
        amask.append(jnp.where(pad, 0.0, 1.0))
        smask.append(jnp.where(pad, -9e8, 0.0))

    s1 = _mm(af, a0w1_ref[...])
    b0 = a0b_ref[0, 0]
    sc = [_leaky(s1 + s2[j] + b0) + smask[j] for j in range(NBR)]
    mx = functools.reduce(jnp.maximum, sc)
    e = [jnp.exp(sc[j] - mx) for j in range(NBR)]
    z = functools.reduce(jnp.add, e)
    attn = [e[j] / z * amask[j] for j in range(NBR)]
    ws = functools.reduce(jnp.add, [attn[j] * nf[j] for j in range(NBR)])
    wsum = functools.reduce(jnp.add, attn)
    ctx = _elu(_mm(ws, t0w_ref[...]) + wsum * t0b_ref[...])

    h1 = _gru(ctx, af,
              g0xr_ref[...], g0xz_ref[...], g0xn_ref[...],
              g0hr_ref[...], g0hz_ref[...], g0hn_ref[...],
              g0bxr_ref[...], g0bxz_ref[...], g0bxn_ref[...],
              g0bhr_ref[...], g0bhz_ref[...], g0bhn_ref[...])
    act = jnp.maximum(h1, 0.0)
    act3 = act.reshape(mb, L, 200)

    s1b = _mm(act, a1w1_ref[...])
    u = _mm(act, a1w2_ref[...])
    u3 = u.reshape(mb, L, 1)
    b1 = a1b_ref[0, 0]
    sc2 = []
    for j in range(NBR):
        su_j = _bmm(oha[j].reshape(mb, L, L), u3).reshape(R, 1)
        sc2.append(_leaky(s1b + su_j + b1) + smask[j])
    mx2 = functools.reduce(jnp.maximum, sc2)
    e2 = [jnp.exp(sc2[j] - mx2) for j in range(NBR)]
    z2 = functools.reduce(jnp.add, e2)
    attn2 = [e2[j] / z2 * amask[j] for j in range(NBR)]
    A2 = functools.reduce(
        jnp.add, [attn2[j] * oha[j] for j in range(NBR)])
    ws2 = _bmm(A2.reshape(mb, L, L), act3).reshape(R, 200)
    wsum2 = functools.reduce(jnp.add, attn2)
    ctx2 = _elu(_mm(ws2, t1w_ref[...]) + wsum2 * t1b_ref[...])

    h2 = _gru(ctx2, h1,
              g1xr_ref[...], g1xz_ref[...], g1xn_ref[...],
              g1hr_ref[...], g1hz_ref[...], g1hn_ref[...],
              g1bxr_ref[...], g1bxz_ref[...], g1bxn_ref[...],
              g1bhr_ref[...], g1bhz_ref[...], g1bhn_ref[...])
    af_out_ref[...] = h2.reshape(mb, L, 200)

    act2 = jnp.maximum(h2, 0.0)
    act2_3 = act2.reshape(mb, L, 200)
    molfeat = jnp.sum((act2 * mask).reshape(mb, L, 200), axis=1)
    mmask_s = jnp.where(mask == 0, -9e8, 0.0)
    s2m = _mm(act2, mw2_ref[...])
    bm = mb_ref[0, 0]
    am = jnp.maximum(molfeat, 0.0)

    for _ in range(2):
        s1m = _mm(am, mw1_ref[...])
        s1m_b = jnp.broadcast_to(s1m.reshape(mb, 1, 1), (mb, L, 1)).reshape(R, 1)
        scm = _leaky(s1m_b + s2m + bm) + mmask_s
        scm3 = scm.reshape(mb, L, 1)
        mxm = jnp.max(scm3, axis=1, keepdims=True)
        em = jnp.exp(scm3 - mxm)
        zm = jnp.sum(em, axis=1, keepdims=True)
        attnm = em / zm * mask.reshape(mb, L, 1)
        wsm = jnp.sum(attnm * act2_3, axis=1)
        wsumm = jnp.sum(attnm, axis=1)
        ctxm = _elu(_mm(wsm, mtw_ref[...]) + wsumm * mtb_ref[...])
        molfeat = _gru(ctxm, molfeat,
                       mgxr_ref[...], mgxz_ref[...], mgxn_ref[...],
                       mghr_ref[...], mghz_ref[...], mghn_ref[...],
                       mgbxr_ref[...], mgbxz_ref[...], mgbxn_ref[...],
                       mgbhr_ref[...], mgbhz_ref[...], mgbhn_ref[...])
        am = jnp.maximum(molfeat, 0.0)

    pred_out_ref[...] = _mm(molfeat, ow_ref[...]) + ob_ref[...]
    molfeat_out_ref[...] = molfeat


def kernel(atom_list, bond_list, atom_degree_list, bond_degree_list, atom_mask, params):
    Bz, L, AD = atom_list.shape
    _, NB, BD = bond_list.shape
    NBR = atom_degree_list.shape[-1]
    FP = params["atom_fc"]["W"].shape[0]
    mb = MB
    grid = Bz // mb

    f32 = jnp.float32
    aidx = atom_degree_list.astype(jnp.int32).reshape(Bz * L, NBR)
    bidx = bond_degree_list.astype(jnp.int32).reshape(Bz * L, NBR)
    mask = atom_mask.astype(f32).reshape(Bz * L, 1)

    def lin_w(p):
        return p["W"].T.astype(f32)

    def row(b):
        return b.reshape(1, -1).astype(f32)

    def gru_parts(g):
        Wih, Whh = g["Wih"], g["Whh"]
        bih, bhh = g["bih"], g["bhh"]
        outs = []
        for W in (Wih, Whh):
            for k in range(3):
                outs.append(W[k * FP:(k + 1) * FP].T.astype(f32))
        for b in (bih, bhh):
            for k in range(3):
                outs.append(b[k * FP:(k + 1) * FP].reshape(1, FP).astype(f32))
        return outs

    def align_parts(a):
        W = a["W"].astype(f32)
        return [W[:, :FP].T, W[:, FP:].T, a["b"].reshape(1, 1).astype(f32)]

    wn = params["neighbor_fc"]["W"].astype(f32)
    weights = (
        [lin_w(params["atom_fc"]), row(params["atom_fc"]["b"]),
         wn[:, :AD].T, wn[:, AD:].T, row(params["neighbor_fc"]["b"])]
        + align_parts(params["align"][0]) + align_parts(params["align"][1])
        + [lin_w(params["attend"][0]), row(params["attend"][0]["b"]),
           lin_w(params["attend"][1]), row(params["attend"][1]["b"])]
        + gru_parts(params["gru"][0]) + gru_parts(params["gru"][1])
        + align_parts(params["mol_align"])
        + [lin_w(params["mol_attend"]), row(params["mol_attend"]["b"])]
        + gru_parts(params["mol_gru"])
        + [lin_w(params["output"]), row(params["output"]["b"])]
    )

    R = mb * L
    data_specs = [
        pl.BlockSpec((mb, L, AD), lambda i: (i, 0, 0)),
        pl.BlockSpec((mb, NB, BD), lambda i: (i, 0, 0)),
        pl.BlockSpec((R, NBR), lambda i: (i, 0)),
        pl.BlockSpec((R, NBR), lambda i: (i, 0)),
        pl.BlockSpec((R, 1), lambda i: (i, 0)),
    ]
    w_specs = [pl.BlockSpec(w.shape, lambda i: tuple(0 for _ in w.shape))
               for w in weights]

    out_shapes = (
        jax.ShapeDtypeStruct((Bz, L, FP), f32),
        jax.ShapeDtypeStruct((Bz, 1), f32),
        jax.ShapeDtypeStruct((Bz, FP), f32),
    )
    out_specs = (
        pl.BlockSpec((mb, L, FP), lambda i: (i, 0, 0)),
        pl.BlockSpec((mb, 1), lambda i: (i, 0)),
        pl.BlockSpec((mb, FP), lambda i: (i, 0)),
    )

    body = functools.partial(_fused_body, mb=mb, L=L, NBR=NBR, NB=NB)
    af, pred, molfeat = pl.pallas_call(
        body,
        grid=(grid,),
        in_specs=data_specs + w_specs,
        out_specs=out_specs,
        out_shape=out_shapes,
        compiler_params=pltpu.CompilerParams(
            dimension_semantics=("arbitrary",),
        ),
    )(atom_list.astype(f32), bond_list.astype(f32), aidx, bidx, mask, *weights)
    return (af, pred, molfeat)


# speedup vs baseline: 22.7273x; 1.1067x over previous
import functools

import jax
import jax.numpy as jnp
from jax.experimental import pallas as pl
from jax.experimental.pallas import tpu as pltpu

MB = 16


def _leaky(x):
    return jnp.maximum(x, 0.01 * x)


def _elu(x):
    return jnp.where(x > 0, x, jnp.exp(x) - 1.0)


def _bmm(a, b):
    return jax.lax.dot_general(
        a.astype(jnp.bfloat16), b.astype(jnp.bfloat16),
        (((2,), (1,)), ((0,), (0,))),
        preferred_element_type=jnp.float32)


def _mm(a, b):
    return jnp.dot(a.astype(jnp.bfloat16), b.astype(jnp.bfloat16),
                   preferred_element_type=jnp.float32)


def _gru(x, h, wxr, wxz, wxn, whr, whz, whn, bxr, bxz, bxn, bhr, bhz, bhn):
    r = jax.nn.sigmoid(_mm(x, wxr) + bxr + _mm(h, whr) + bhr)
    z = jax.nn.sigmoid(_mm(x, wxz) + bxz + _mm(h, whz) + bhz)
    n = jnp.tanh(_mm(x, wxn) + bxn + r * (_mm(h, whn) + bhn))
    return (1.0 - z) * n + z * h


def _fused_body(
    atom_ref, bond_ref, aidx_ref, bidx_ref, mask_ref,
    wf_ref, bf_ref, wna_ref, wnb_ref, bn_ref,
    a0w1_ref, a0w2_ref, a0b_ref, a1w1_ref, a1w2_ref, a1b_ref,
    t0w_ref, t0b_ref, t1w_ref, t1b_ref,
    g0xr_ref, g0xz_ref, g0xn_ref, g0hr_ref, g0hz_ref, g0hn_ref,
    g0bxr_ref, g0bxz_ref, g0bxn_ref, g0bhr_ref, g0bhz_ref, g0bhn_ref,
    g1xr_ref, g1xz_ref, g1xn_ref, g1hr_ref, g1hz_ref, g1hn_ref,
    g1bxr_ref, g1bxz_ref, g1bxn_ref, g1bhr_ref, g1bhz_ref, g1bhn_ref,
    mw1_ref, mw2_ref, mb_ref, mtw_ref, mtb_ref,
    mgxr_ref, mgxz_ref, mgxn_ref, mghr_ref, mghz_ref, mghn_ref,
    mgbxr_ref, mgbxz_ref, mgbxn_ref, mgbhr_ref, mgbhz_ref, mgbhn_ref,
    ow_ref, ob_ref,
    af_out_ref, pred_out_ref, molfeat_out_ref,
    *, mb, L, NBR, NB,
):
    R = mb * L
    atom = atom_ref[...].reshape(R, atom_ref.shape[-1])
    bond = bond_ref[...].reshape(mb * NB, bond_ref.shape[-1])
    aidx = aidx_ref[...]
    bidx = bidx_ref[...]
    mask = mask_ref[...]

    af = _leaky(_mm(atom, wf_ref[...]) + bf_ref[...])
    P = _mm(atom, wna_ref[...])
    Q = _mm(bond, wnb_ref[...])
    P3 = P.reshape(mb, L, 200)
    Q3 = Q.reshape(mb, NB, 200)

    iota_a = jax.lax.broadcasted_iota(jnp.int32, (1, L), 1)
    iota_b = jax.lax.broadcasted_iota(jnp.int32, (1, NB), 1)

    oha = []
    nf = []
    s2 = []
    amask = []
    smask = []
    for j in range(NBR):
        ij = aidx[:, j:j + 1]
        oha_j = (ij == iota_a).astype(jnp.bfloat16)
        ohb_j = (bidx[:, j:j + 1] == iota_b).astype(jnp.bfloat16)
        nfP = _bmm(oha_j.reshape(mb, L, L), P3).reshape(R, 200)
        nfQ = _bmm(ohb_j.reshape(mb, L, NB), Q3).reshape(R, 200)
        nf_j = _leaky(nfP + nfQ + bn_ref[...])
        pad = (ij == (L - 1))
        oha.append(oha_j)
        nf.append(nf_j)
        s2.append(_mm(nf_j, a0w2_ref[...]))
        amask.append(jnp.where(pad, 0.0, 1.0))
        smask.append(jnp.where(pad, -9e8, 0.0))

    s1 = _mm(af, a0w1_ref[...])
    b0 = a0b_ref[0, 0]
    sc = [_leaky(s1 + s2[j] + b0) + smask[j] for j in range(NBR)]
    mx = functools.reduce(jnp.maximum, sc)
    e = [jnp.exp(sc[j] - mx) for j in range(NBR)]
    z = functools.reduce(jnp.add, e)
    attn = [e[j] / z * amask[j] for j in range(NBR)]
    ws = functools.reduce(jnp.add, [attn[j] * nf[j] for j in range(NBR)])
    wsum = functools.reduce(jnp.add, attn)
    ctx = _elu(_mm(ws, t0w_ref[...]) + wsum * t0b_ref[...])

    h1 = _gru(ctx, af,
              g0xr_ref[...], g0xz_ref[...], g0xn_ref[...],
              g0hr_ref[...], g0hz_ref[...], g0hn_ref[...],
              g0bxr_ref[...], g0bxz_ref[...], g0bxn_ref[...],
              g0bhr_ref[...], g0bhz_ref[...], g0bhn_ref[...])
    act = jnp.maximum(h1, 0.0)
    act3 = act.reshape(mb, L, 200)

    s1b = _mm(act, a1w1_ref[...])
    u = _mm(act, a1w2_ref[...])
    u3 = u.reshape(mb, L, 1)
    b1 = a1b_ref[0, 0]
    sc2 = []
    for j in range(NBR):
        su_j = _bmm(oha[j].reshape(mb, L, L), u3).reshape(R, 1)
        sc2.append(_leaky(s1b + su_j + b1) + smask[j])
    mx2 = functools.reduce(jnp.maximum, sc2)
    e2 = [jnp.exp(sc2[j] - mx2) for j in range(NBR)]
    z2 = functools.reduce(jnp.add, e2)
    attn2 = [e2[j] / z2 * amask[j] for j in range(NBR)]
    A2 = functools.reduce(
        jnp.add, [attn2[j] * oha[j] for j in range(NBR)])
    ws2 = _bmm(A2.reshape(mb, L, L), act3).reshape(R, 200)
    wsum2 = functools.reduce(jnp.add, attn2)
    ctx2 = _elu(_mm(ws2, t1w_ref[...]) + wsum2 * t1b_ref[...])

    h2 = _gru(ctx2, h1,
              g1xr_ref[...], g1xz_ref[...], g1xn_ref[...],
              g1hr_ref[...], g1hz_ref[...], g1hn_ref[...],
              g1bxr_ref[...], g1bxz_ref[...], g1bxn_ref[...],
              g1bhr_ref[...], g1bhz_ref[...], g1bhn_ref[...])
    af_out_ref[...] = h2.reshape(mb, L, 200)

    act2 = jnp.maximum(h2, 0.0)
    act2_3 = act2.reshape(mb, L, 200)
    molfeat = jnp.sum((act2 * mask).reshape(mb, L, 200), axis=1)
    mmask_s = jnp.where(mask == 0, -9e8, 0.0)
    s2m = _mm(act2, mw2_ref[...])
    bm = mb_ref[0, 0]
    am = jnp.maximum(molfeat, 0.0)

    for _ in range(2):
        s1m = _mm(am, mw1_ref[...])
        s1m_b = jnp.broadcast_to(s1m.reshape(mb, 1, 1), (mb, L, 1)).reshape(R, 1)
        scm = _leaky(s1m_b + s2m + bm) + mmask_s
        scm3 = scm.reshape(mb, L, 1)
        mxm = jnp.max(scm3, axis=1, keepdims=True)
        em = jnp.exp(scm3 - mxm)
        zm = jnp.sum(em, axis=1, keepdims=True)
        attnm = em / zm * mask.reshape(mb, L, 1)
        wsm = jnp.sum(attnm * act2_3, axis=1)
        wsumm = jnp.sum(attnm, axis=1)
        ctxm = _elu(_mm(wsm, mtw_ref[...]) + wsumm * mtb_ref[...])
        molfeat = _gru(ctxm, molfeat,
                       mgxr_ref[...], mgxz_ref[...], mgxn_ref[...],
                       mghr_ref[...], mghz_ref[...], mghn_ref[...],
                       mgbxr_ref[...], mgbxz_ref[...], mgbxn_ref[...],
                       mgbhr_ref[...], mgbhz_ref[...], mgbhn_ref[...])
        am = jnp.maximum(molfeat, 0.0)

    pred_out_ref[...] = _mm(molfeat, ow_ref[...]) + ob_ref[...]
    molfeat_out_ref[...] = molfeat


def kernel(atom_list, bond_list, atom_degree_list, bond_degree_list, atom_mask, params):
    Bz, L, AD = atom_list.shape
    _, NB, BD = bond_list.shape
    NBR = atom_degree_list.shape[-1]
    FP = params["atom_fc"]["W"].shape[0]
    mb = MB
    grid = Bz // mb

    f32 = jnp.float32
    aidx = atom_degree_list.astype(jnp.int32).reshape(Bz * L, NBR)
    bidx = bond_degree_list.astype(jnp.int32).reshape(Bz * L, NBR)
    mask = atom_mask.astype(f32).reshape(Bz * L, 1)

    def lin_w(p):
        return p["W"].T.astype(f32)

    def row(b):
        return b.reshape(1, -1).astype(f32)

    def gru_parts(g):
        Wih, Whh = g["Wih"], g["Whh"]
        bih, bhh = g["bih"], g["bhh"]
        outs = []
        for W in (Wih, Whh):
            for k in range(3):
                outs.append(W[k * FP:(k + 1) * FP].T.astype(f32))
        for b in (bih, bhh):
            for k in range(3):
                outs.append(b[k * FP:(k + 1) * FP].reshape(1, FP).astype(f32))
        return outs

    def align_parts(a):
        W = a["W"].astype(f32)
        return [W[:, :FP].T, W[:, FP:].T, a["b"].reshape(1, 1).astype(f32)]

    wn = params["neighbor_fc"]["W"].astype(f32)
    weights = (
        [lin_w(params["atom_fc"]), row(params["atom_fc"]["b"]),
         wn[:, :AD].T, wn[:, AD:].T, row(params["neighbor_fc"]["b"])]
        + align_parts(params["align"][0]) + align_parts(params["align"][1])
        + [lin_w(params["attend"][0]), row(params["attend"][0]["b"]),
           lin_w(params["attend"][1]), row(params["attend"][1]["b"])]
        + gru_parts(params["gru"][0]) + gru_parts(params["gru"][1])
        + align_parts(params["mol_align"])
        + [lin_w(params["mol_attend"]), row(params["mol_attend"]["b"])]
        + gru_parts(params["mol_gru"])
        + [lin_w(params["output"]), row(params["output"]["b"])]
    )

    R = mb * L
    data_specs = [
        pl.BlockSpec((mb, L, AD), lambda i: (i, 0, 0)),
        pl.BlockSpec((mb, NB, BD), lambda i: (i, 0, 0)),
        pl.BlockSpec((R, NBR), lambda i: (i, 0)),
        pl.BlockSpec((R, NBR), lambda i: (i, 0)),
        pl.BlockSpec((R, 1), lambda i: (i, 0)),
    ]
    w_specs = [pl.BlockSpec(w.shape, lambda i: tuple(0 for _ in w.shape))
               for w in weights]

    out_shapes = (
        jax.ShapeDtypeStruct((Bz, L, FP), f32),
        jax.ShapeDtypeStruct((Bz, 1), f32),
        jax.ShapeDtypeStruct((Bz, FP), f32),
    )
    out_specs = (
        pl.BlockSpec((mb, L, FP), lambda i: (i, 0, 0)),
        pl.BlockSpec((mb, 1), lambda i: (i, 0)),
        pl.BlockSpec((mb, FP), lambda i: (i, 0)),
    )

    body = functools.partial(_fused_body, mb=mb, L=L, NBR=NBR, NB=NB)
    af, pred, molfeat = pl.pallas_call(
        body,
        grid=(grid,),
        in_specs=data_specs + w_specs,
        out_specs=out_specs,
        out_shape=out_shapes,
        compiler_params=pltpu.CompilerParams(
            dimension_semantics=("arbitrary",),
        ),
    )(atom_list.astype(f32), bond_list.astype(f32), aidx, bidx, mask, *weights)
    return (af, pred, molfeat)
